# baseline (speedup 1.0000x reference)
import jax
import jax.numpy as jnp
from jax.experimental import pallas as pl
from jax.experimental.pallas import tpu as pltpu


def _make_fused_kernel(tb, E, N, M, O, first_steps):
    def _fused(x_ref, f_hbm, w_hbm, g_hbm, o_ref, f_v, w_v, g_v, sem):
        i = pl.program_id(0)

        cond = i == first_steps[0]
        for s in first_steps[1:]:
            cond = jnp.logical_or(cond, i == s)

        @pl.when(cond)
        def _load_invariants():
            for src, dst in ((f_hbm, f_v), (w_hbm, w_v), (g_hbm, g_v)):
                cp = pltpu.make_async_copy(src, dst, sem)
                cp.start()
                cp.wait()

        spec = jnp.dot(x_ref[...].reshape(tb * E, N), f_v[...],
                       preferred_element_type=jnp.float32)
        st = jnp.transpose(spec.reshape(tb, E, 2 * M), (0, 2, 1))
        x2 = st.reshape(tb, 2, M, E).transpose(2, 0, 1, 3).reshape(
            M, tb, 2 * E)
        d = jnp.stack([jnp.dot(x2[m], w_v[m],
                               preferred_element_type=jnp.float32)
                       for m in range(M)], axis=0)
        coef_t = d.reshape(M, tb, 2, O).transpose(2, 0, 1, 3).reshape(
            2 * M, tb * O)
        y = jax.lax.dot_general(
            coef_t, g_v[...], (((0,), (0,)), ((), ())),
            preferred_element_type=jnp.float32)
        o_ref[...] = y.reshape(tb, O, N)
    return _fused


def _pick_tile(rows, target):
    tm = min(target, rows)
    while rows % tm:
        tm -= 1
    return tm


@jax.jit
def kernel(x, weights_r, weights_i):
    B, H, E, N = x.shape
    _, O, M = weights_r.shape
    BH = B * H

    n_idx = jnp.arange(N, dtype=jnp.float32)[:, None]
    m_idx = jnp.arange(M, dtype=jnp.float32)[None, :]
    ang = (2.0 * jnp.pi / N) * n_idx * m_idx
    fwd = jnp.concatenate([jnp.cos(ang), -jnp.sin(ang)], axis=1)

    scale = jnp.where(jnp.arange(M) == 0, 1.0, 2.0)[:, None] / N
    inv = jnp.concatenate([scale * jnp.cos(ang.T),
                           -scale * jnp.sin(ang.T)], axis=0)

    wrm = jnp.transpose(weights_r, (2, 0, 1)).astype(jnp.float32)
    wim = jnp.transpose(weights_i, (2, 0, 1)).astype(jnp.float32)
    w_mix = jnp.concatenate([jnp.concatenate([wrm, wim], 2),
                             jnp.concatenate([-wim, wrm], 2)], 1)

    tb = _pick_tile(BH, 32)
    grid = BH // tb
    first_steps = sorted({0, grid // 2})
    y = pl.pallas_call(
        _make_fused_kernel(tb, E, N, M, O, tuple(first_steps)),
        out_shape=jax.ShapeDtypeStruct((BH, O, N), jnp.float32),
        grid=(grid,),
        in_specs=[
            pl.BlockSpec((tb, E, N), lambda i: (i, 0, 0)),
            pl.BlockSpec(memory_space=pl.ANY),
            pl.BlockSpec(memory_space=pl.ANY),
            pl.BlockSpec(memory_space=pl.ANY),
        ],
        out_specs=pl.BlockSpec((tb, O, N), lambda i: (i, 0, 0)),
        scratch_shapes=[
            pltpu.VMEM((N, 2 * M), jnp.float32),
            pltpu.VMEM((M, 2 * E, 2 * O), jnp.float32),
            pltpu.VMEM((2 * M, N), jnp.float32),
            pltpu.SemaphoreType.DMA,
        ],
        compiler_params=pltpu.CompilerParams(
            dimension_semantics=("parallel",)),
    )(x.reshape(BH, E, N), fwd, w_mix, inv)
    return y.reshape(B, H, O, N)


# speedup vs baseline: 1.0973x; 1.0973x over previous
import functools

import jax
import jax.numpy as jnp
from jax.experimental import pallas as pl
from jax.experimental.pallas import tpu as pltpu


def _make_fused_kernel(tb, E, N, M, O):
    def _fused(x_ref, f_ref, w_ref, g_ref, o_ref):
        spec = jnp.dot(x_ref[...].reshape(tb * E, N), f_ref[...],
                       preferred_element_type=jnp.float32)
        st = jnp.transpose(spec.reshape(tb, E, 2 * M), (0, 2, 1))
        x2 = st.reshape(tb, 2, M, E).transpose(2, 0, 1, 3).reshape(
            M, tb, 2 * E)
        x2b = x2.astype(jnp.bfloat16)
        d = jnp.stack([jnp.dot(x2b[m], w_ref[m],
                               preferred_element_type=jnp.float32)
                       for m in range(M)], axis=0)
        coef_t = d.reshape(M, tb, 2, O).transpose(2, 0, 1, 3).reshape(
            2 * M, tb * O)
        y = jax.lax.dot_general(
            coef_t.astype(jnp.bfloat16), g_ref[...],
            (((0,), (0,)), ((), ())),
            preferred_element_type=jnp.float32)
        o_ref[...] = y.reshape(tb, O, N)
    return _fused


def _pick_tile(rows, target):
    tm = min(target, rows)
    while rows % tm:
        tm -= 1
    return tm


@jax.jit
def kernel(x, weights_r, weights_i):
    B, H, E, N = x.shape
    _, O, M = weights_r.shape
    BH = B * H

    n_idx = jnp.arange(N, dtype=jnp.float32)[:, None]
    m_idx = jnp.arange(M, dtype=jnp.float32)[None, :]
    ang = (2.0 * jnp.pi / N) * n_idx * m_idx
    fwd = jnp.concatenate([jnp.cos(ang), -jnp.sin(ang)], axis=1)

    scale = jnp.where(jnp.arange(M) == 0, 1.0, 2.0)[:, None] / N
    inv = jnp.concatenate([scale * jnp.cos(ang.T),
                           -scale * jnp.sin(ang.T)],
                          axis=0).astype(jnp.bfloat16)

    wrm = jnp.transpose(weights_r, (2, 0, 1)).astype(jnp.float32)
    wim = jnp.transpose(weights_i, (2, 0, 1)).astype(jnp.float32)
    w_mix = jnp.concatenate([jnp.concatenate([wrm, wim], 2),
                             jnp.concatenate([-wim, wrm], 2)],
                            1).astype(jnp.bfloat16)

    tb = _pick_tile(BH, 32)
    y = pl.pallas_call(
        _make_fused_kernel(tb, E, N, M, O),
        out_shape=jax.ShapeDtypeStruct((BH, O, N), jnp.float32),
        grid=(BH // tb,),
        in_specs=[
            pl.BlockSpec((tb, E, N), lambda i: (i, 0, 0)),
            pl.BlockSpec((N, 2 * M), lambda i: (0, 0)),
            pl.BlockSpec((M, 2 * E, 2 * O), lambda i: (0, 0, 0)),
            pl.BlockSpec((2 * M, N), lambda i: (0, 0)),
        ],
        out_specs=pl.BlockSpec((tb, O, N), lambda i: (i, 0, 0)),
        compiler_params=pltpu.CompilerParams(
            dimension_semantics=("parallel",)),
    )(x.reshape(BH, E, N), fwd, w_mix, inv)
    return y.reshape(B, H, O, N)


# speedup vs baseline: 1.1137x; 1.0150x over previous
import jax
import jax.numpy as jnp
from jax.experimental import pallas as pl
from jax.experimental.pallas import tpu as pltpu


def _make_fused_kernel(tb, E, N, M, O):
    def _fused(x_ref, f_ref, w_ref, g_ref, o_ref):
        spec = jnp.dot(x_ref[...].reshape(tb * E, N), f_ref[...],
                       preferred_element_type=jnp.float32)
        st = jnp.transpose(spec.reshape(tb, E, 2 * M), (0, 2, 1))
        x2 = st.reshape(tb, 2, M, E).transpose(2, 0, 1, 3).reshape(
            M, tb, 2 * E)
        d = jnp.stack([jnp.dot(x2[m], w_ref[m],
                               preferred_element_type=jnp.float32)
                       for m in range(M)], axis=0)
        coef_t = d.reshape(M, tb, 2, O).transpose(2, 0, 1, 3).reshape(
            2 * M, tb * O)
        y = jax.lax.dot_general(
            coef_t, g_ref[...], (((0,), (0,)), ((), ())),
            preferred_element_type=jnp.float32)
        o_ref[...] = y.reshape(tb, O, N)
    return _fused


def _pick_tile(rows, target):
    tm = min(target, rows)
    while rows % tm:
        tm -= 1
    return tm


@jax.jit
def kernel(x, weights_r, weights_i):
    B, H, E, N = x.shape
    _, O, M = weights_r.shape
    BH = B * H

    n_idx = jnp.arange(N, dtype=jnp.float32)[:, None]
    m_idx = jnp.arange(M, dtype=jnp.float32)[None, :]
    ang = (2.0 * jnp.pi / N) * n_idx * m_idx
    fwd = jnp.concatenate([jnp.cos(ang), -jnp.sin(ang)], axis=1)

    scale = jnp.where(jnp.arange(M) == 0, 1.0, 2.0)[:, None] / N
    inv = jnp.concatenate([scale * jnp.cos(ang.T),
                           -scale * jnp.sin(ang.T)], axis=0)

    wrm = jnp.transpose(weights_r, (2, 0, 1)).astype(jnp.float32)
    wim = jnp.transpose(weights_i, (2, 0, 1)).astype(jnp.float32)
    w_mix = jnp.concatenate([jnp.concatenate([wrm, wim], 2),
                             jnp.concatenate([-wim, wrm], 2)], 1)

    tb = _pick_tile(BH, 32)
    y = pl.pallas_call(
        _make_fused_kernel(tb, E, N, M, O),
        out_shape=jax.ShapeDtypeStruct((BH, O, N), jnp.float32),
        grid=(BH // tb,),
        in_specs=[
            pl.BlockSpec((tb, E, N), lambda i: (i, 0, 0)),
            pl.BlockSpec((N, 2 * M), lambda i: (0, 0)),
            pl.BlockSpec((M, 2 * E, 2 * O), lambda i: (0, 0, 0)),
            pl.BlockSpec((2 * M, N), lambda i: (0, 0)),
        ],
        out_specs=pl.BlockSpec((tb, O, N), lambda i: (i, 0, 0)),
        compiler_params=pltpu.CompilerParams(
            dimension_semantics=("parallel",)),
    )(x.reshape(BH, E, N), fwd, w_mix, inv)
    return y.reshape(B, H, O, N)
